# baseline (speedup 1.0000x reference)
import functools

import jax
import jax.numpy as jnp
from jax import lax
from jax.experimental import pallas as pl
from jax.experimental.pallas import tpu as pltpu
from jax.experimental.pallas import tpu_sc as plsc

E = 320000
D = 128
T1 = 6
T2 = 3
TC_ = T1 * T2
TCP = 24
NC = 2
NS = 16
NW = NC * NS
BPW = E // NW
CHUNK = 200
NCHUNK = BPW // CHUNK
NBUF = 3
LANES = 16


def _body(a0_hbm, a1_hbm, t1_hbm, t2_hbm, out_hbm,
          t1_v, t2_v, comb_v, ctab_sh, a0_v, a1_v, rows0, rows1, rows2,
          gsem0, gsem1, gsem2, ssem0, ssem1, ssem2):
    rows = (rows0, rows1, rows2)
    gsem = (gsem0, gsem1, gsem2)
    ssem = (ssem0, ssem1, ssem2)
    sid = lax.axis_index("s")
    wid = sid * NC + lax.axis_index("c")
    base0 = wid * BPW

    pltpu.sync_copy(t1_hbm, t1_v)
    pltpu.sync_copy(t2_hbm, t2_v)
    for i in range(T1):
        for j in range(T2):
            for k in range(D // LANES):
                comb_v[i * T2 + j, pl.ds(k * LANES, LANES)] = (
                    t1_v[i, pl.ds(k * LANES, LANES)]
                    + t2_v[j, pl.ds(k * LANES, LANES)])
    zf = jnp.zeros((LANES,), jnp.float32)
    for i in range(TC_, TCP):
        for k in range(D // LANES):
            comb_v[i, pl.ds(k * LANES, LANES)] = zf
    @pl.when(sid == 0)
    def _():
        pltpu.sync_copy(comb_v, ctab_sh)
    plsc.subcore_barrier()

    cp0 = pltpu.async_copy(a0_hbm.at[pl.ds(base0, BPW)], a0_v, gsem0)
    cp1 = pltpu.async_copy(a1_hbm.at[pl.ds(base0, BPW)], a1_v, gsem1)
    cp0.wait()
    cp1.wait()

    def idx_body(g, carry):
        sl = pl.ds(pl.multiple_of(g * LANES, LANES), LANES)
        a0_v[sl] = a0_v[sl] * 3 + a1_v[sl]
        return carry

    lax.fori_loop(0, BPW // LANES, idx_body, 0)

    def gather(c, b):
        return pltpu.async_copy(
            ctab_sh.at[a0_v.at[pl.ds(c * CHUNK, CHUNK)]], rows[b], gsem[b])

    scats = [None] * NBUF
    gaths = [None] * NBUF
    gaths[0] = gather(0, 0)
    for c in range(NCHUNK):
        b = c % NBUF
        if c + 1 < NCHUNK:
            nb = (c + 1) % NBUF
            if scats[nb] is not None:
                scats[nb].wait()
                scats[nb] = None
            gaths[nb] = gather(c + 1, nb)
        gaths[b].wait()
        scats[b] = pltpu.async_copy(
            rows[b], out_hbm.at[pl.ds(base0 + c * CHUNK, CHUNK)], ssem[b])
    for b in range(NBUF):
        if scats[b] is not None:
            scats[b].wait()


def kernel(edge_attr, table1, table2):
    mesh = plsc.VectorSubcoreMesh(core_axis_name="c", subcore_axis_name="s")
    kfn = functools.partial(
        pl.kernel,
        out_type=jax.ShapeDtypeStruct((E, D), jnp.float32),
        mesh=mesh,
        scratch_types=[
            pltpu.VMEM((T1, D), jnp.float32),
            pltpu.VMEM((T2, D), jnp.float32),
            pltpu.VMEM((TCP, D), jnp.float32),
            pltpu.VMEM_SHARED((TCP, D), jnp.float32),
            pltpu.VMEM((BPW,), jnp.int32),
            pltpu.VMEM((BPW,), jnp.int32),
            pltpu.VMEM((CHUNK, D), jnp.float32),
            pltpu.VMEM((CHUNK, D), jnp.float32),
            pltpu.VMEM((CHUNK, D), jnp.float32),
            pltpu.SemaphoreType.DMA,
            pltpu.SemaphoreType.DMA,
            pltpu.SemaphoreType.DMA,
            pltpu.SemaphoreType.DMA,
            pltpu.SemaphoreType.DMA,
            pltpu.SemaphoreType.DMA,
        ],
    )(_body)
    a0 = edge_attr[:, 0]
    a1 = edge_attr[:, 1]
    return kfn(a0, a1, table1, table2)


# speedup vs baseline: 1.0687x; 1.0687x over previous
import functools

import jax
import jax.numpy as jnp
from jax import lax
from jax.experimental import pallas as pl
from jax.experimental.pallas import tpu as pltpu
from jax.experimental.pallas import tpu_sc as plsc

E = 320000
D = 128
T1 = 6
T2 = 3
TC_ = T1 * T2
TCP = 24
NC = 2
NS = 16
NW = NC * NS
BPW = E // NW
CHUNK = 400
NCHUNK = BPW // CHUNK
NBUF = 2
LANES = 16
UNROLL = 4


def _body(a0_hbm, a1_hbm, t1_hbm, t2_hbm, out_hbm,
          t1_v, t2_v, comb_v, ctab_sh, a0_v, a1_v, rows0, rows1,
          gsem0, gsem1, ssem0, ssem1):
    rows = (rows0, rows1)
    gsem = (gsem0, gsem1)
    ssem = (ssem0, ssem1)
    sid = lax.axis_index("s")
    wid = sid * NC + lax.axis_index("c")
    base0 = wid * BPW

    cp0 = pltpu.async_copy(a0_hbm.at[pl.ds(base0, BPW)], a0_v, gsem0)
    cp1 = pltpu.async_copy(a1_hbm.at[pl.ds(base0, BPW)], a1_v, gsem1)

    pltpu.sync_copy(t1_hbm, t1_v)
    pltpu.sync_copy(t2_hbm, t2_v)
    for i in range(T1):
        for j in range(T2):
            for k in range(D // LANES):
                comb_v[i * T2 + j, pl.ds(k * LANES, LANES)] = (
                    t1_v[i, pl.ds(k * LANES, LANES)]
                    + t2_v[j, pl.ds(k * LANES, LANES)])
    zf = jnp.zeros((LANES,), jnp.float32)
    for i in range(TC_, TCP):
        for k in range(D // LANES):
            comb_v[i, pl.ds(k * LANES, LANES)] = zf
    @pl.when(sid == 0)
    def _():
        pltpu.sync_copy(comb_v, ctab_sh)
    plsc.subcore_barrier()

    cp0.wait()
    cp1.wait()

    def idx_body(g, carry):
        for u in range(UNROLL):
            sl = pl.ds(pl.multiple_of(g * (LANES * UNROLL), LANES)
                       + u * LANES, LANES)
            a0_v[sl] = a0_v[sl] * 3 + a1_v[sl]
        return carry

    lax.fori_loop(0, BPW // (LANES * UNROLL), idx_body, 0)

    scats = [None] * NBUF
    for c in range(NCHUNK):
        b = c % NBUF
        if scats[b] is not None:
            scats[b].wait()
        pltpu.async_copy(
            ctab_sh.at[a0_v.at[pl.ds(c * CHUNK, CHUNK)]], rows[b], gsem[b]
        ).wait()
        scats[b] = pltpu.async_copy(
            rows[b], out_hbm.at[pl.ds(base0 + c * CHUNK, CHUNK)], ssem[b])
    for b in range(NBUF):
        if scats[b] is not None:
            scats[b].wait()


def kernel(edge_attr, table1, table2):
    mesh = plsc.VectorSubcoreMesh(core_axis_name="c", subcore_axis_name="s")
    kfn = functools.partial(
        pl.kernel,
        out_type=jax.ShapeDtypeStruct((E, D), jnp.float32),
        mesh=mesh,
        scratch_types=[
            pltpu.VMEM((T1, D), jnp.float32),
            pltpu.VMEM((T2, D), jnp.float32),
            pltpu.VMEM((TCP, D), jnp.float32),
            pltpu.VMEM_SHARED((TCP, D), jnp.float32),
            pltpu.VMEM((BPW,), jnp.int32),
            pltpu.VMEM((BPW,), jnp.int32),
            pltpu.VMEM((CHUNK, D), jnp.float32),
            pltpu.VMEM((CHUNK, D), jnp.float32),
            pltpu.SemaphoreType.DMA,
            pltpu.SemaphoreType.DMA,
            pltpu.SemaphoreType.DMA,
            pltpu.SemaphoreType.DMA,
        ],
    )(_body)
    a0 = edge_attr[:, 0]
    a1 = edge_attr[:, 1]
    return kfn(a0, a1, table1, table2)


# speedup vs baseline: 1.0730x; 1.0040x over previous
import functools

import jax
import jax.numpy as jnp
from jax import lax
from jax.experimental import pallas as pl
from jax.experimental.pallas import tpu as pltpu
from jax.experimental.pallas import tpu_sc as plsc

E = 320000
D = 128
T1 = 6
T2 = 3
TC_ = T1 * T2
TCP = 24
NC = 2
NS = 16
NW = NC * NS
BPW = E // NW
CHUNK = 400
NCHUNK = BPW // CHUNK
NBUF = 2
LANES = 16
UNROLL = 5


def _body(a0_hbm, a1_hbm, t1_hbm, t2_hbm, out_hbm,
          t1_v, t2_v, comb_v, ctab_sh, a0_v, a1_v, rows0, rows1,
          gsem0, gsem1, ssem0, ssem1):
    rows = (rows0, rows1)
    gsem = (gsem0, gsem1)
    ssem = (ssem0, ssem1)
    sid = lax.axis_index("s")
    wid = sid * NC + lax.axis_index("c")
    base0 = wid * BPW

    cp0 = pltpu.async_copy(a0_hbm.at[pl.ds(base0, BPW)], a0_v, gsem0)
    cp1 = pltpu.async_copy(a1_hbm.at[pl.ds(base0, BPW)], a1_v, gsem1)

    pltpu.sync_copy(t1_hbm, t1_v)
    pltpu.sync_copy(t2_hbm, t2_v)
    for i in range(T1):
        for j in range(T2):
            for k in range(D // LANES):
                comb_v[i * T2 + j, pl.ds(k * LANES, LANES)] = (
                    t1_v[i, pl.ds(k * LANES, LANES)]
                    + t2_v[j, pl.ds(k * LANES, LANES)])
    zf = jnp.zeros((LANES,), jnp.float32)
    for i in range(TC_, TCP):
        for k in range(D // LANES):
            comb_v[i, pl.ds(k * LANES, LANES)] = zf
    @pl.when(sid == 0)
    def _():
        pltpu.sync_copy(comb_v, ctab_sh)
    plsc.subcore_barrier()

    cp0.wait()
    cp1.wait()

    def idx_body(g, carry):
        for u in range(UNROLL):
            sl = pl.ds(pl.multiple_of(g * (LANES * UNROLL), LANES)
                       + u * LANES, LANES)
            a0_v[sl] = a0_v[sl] * 3 + a1_v[sl]
        return carry

    lax.fori_loop(0, BPW // (LANES * UNROLL), idx_body, 0)

    scats = [None] * NBUF
    for c in range(NCHUNK):
        b = c % NBUF
        if scats[b] is not None:
            scats[b].wait()
        pltpu.async_copy(
            ctab_sh.at[a0_v.at[pl.ds(c * CHUNK, CHUNK)]], rows[b], gsem[b]
        ).wait()
        scats[b] = pltpu.async_copy(
            rows[b], out_hbm.at[pl.ds(base0 + c * CHUNK, CHUNK)], ssem[b])
    for b in range(NBUF):
        if scats[b] is not None:
            scats[b].wait()


def kernel(edge_attr, table1, table2):
    mesh = plsc.VectorSubcoreMesh(core_axis_name="c", subcore_axis_name="s")
    kfn = functools.partial(
        pl.kernel,
        out_type=jax.ShapeDtypeStruct((E, D), jnp.float32),
        mesh=mesh,
        scratch_types=[
            pltpu.VMEM((T1, D), jnp.float32),
            pltpu.VMEM((T2, D), jnp.float32),
            pltpu.VMEM((TCP, D), jnp.float32),
            pltpu.VMEM_SHARED((TCP, D), jnp.float32),
            pltpu.VMEM((BPW,), jnp.int32),
            pltpu.VMEM((BPW,), jnp.int32),
            pltpu.VMEM((CHUNK, D), jnp.float32),
            pltpu.VMEM((CHUNK, D), jnp.float32),
            pltpu.SemaphoreType.DMA,
            pltpu.SemaphoreType.DMA,
            pltpu.SemaphoreType.DMA,
            pltpu.SemaphoreType.DMA,
        ],
    )(_body)
    a0 = edge_attr[:, 0]
    a1 = edge_attr[:, 1]
    return kfn(a0, a1, table1, table2)
